# baseline (speedup 1.0000x reference)
import functools

import jax
import jax.numpy as jnp
from jax import lax
from jax.experimental import pallas as pl
from jax.experimental.pallas import tpu as pltpu
from jax.experimental.pallas import tpu_sc as plsc

_TOK = 32768
_D = 128
_T = 4096

_NC = 2
_NS = 16
_NW = _NC * _NS
_BPW = _TOK // _NW
_CH = 128
_NCH = _BPW // _CH
_NTYPES = 100


def _sc_body(z2_hbm, table_hbm, out_hbm, idx_v, rows_a, rows_b, tab_sh,
             bounce, gsem, ssem_a, ssem_b):
    cid = lax.axis_index("c")
    sid = lax.axis_index("s")
    wid = sid * _NC + cid
    base = wid * _BPW
    brow = wid * _NCH

    @pl.when(sid == 0)
    def _():
        pltpu.sync_copy(table_hbm, bounce)
        pltpu.sync_copy(bounce, tab_sh)
    plsc.subcore_barrier()

    pltpu.sync_copy(z2_hbm.at[pl.ds(brow, _NCH)], idx_v)

    bufs = (rows_a, rows_b)
    ssems = (ssem_a, ssem_b)
    stores = [None, None]
    gath = pltpu.async_copy(tab_sh.at[idx_v.at[0]], bufs[0], gsem)
    for i in range(_NCH):
        b = i % 2
        gath.wait()
        st = pltpu.async_copy(
            bufs[b], out_hbm.at[pl.ds(base + i * _CH, _CH)], ssems[b])
        stores[b] = st
        if i + 1 < _NCH:
            nb = (i + 1) % 2
            if stores[nb] is not None:
                stores[nb].wait()
            gath = pltpu.async_copy(
                tab_sh.at[idx_v.at[i + 1]], bufs[nb], gsem)
    stores[(_NCH - 2) % 2].wait()
    stores[(_NCH - 1) % 2].wait()


def _sc_gather(z, table_w):
    mesh = plsc.VectorSubcoreMesh(core_axis_name="c", subcore_axis_name="s")
    return pl.kernel(
        _sc_body,
        mesh=mesh,
        out_type=jax.ShapeDtypeStruct((_TOK, _D // 2), jnp.int32),
        scratch_types=[
            pltpu.VMEM((_NCH, _CH), jnp.int32),
            pltpu.VMEM((_CH, _D // 2), jnp.int32),
            pltpu.VMEM((_CH, _D // 2), jnp.int32),
            pltpu.VMEM_SHARED((_NTYPES, _D // 2), jnp.int32),
            pltpu.VMEM((_NTYPES, _D // 2), jnp.int32),
            pltpu.SemaphoreType.DMA,
            pltpu.SemaphoreType.DMA,
            pltpu.SemaphoreType.DMA,
        ],
    )(z.reshape(_TOK // _CH, _CH), table_w)


def _tc_body(emb_ref, x_ref, W1_ref, b1_ref, W2_ref, b2_ref,
             g_ref, bt_ref, out_ref):
    x = x_ref[...]
    p = jnp.dot(x, W1_ref[...], preferred_element_type=jnp.float32)
    p = p + b1_ref[...]
    p = p * jax.nn.sigmoid(p)
    h = jnp.dot(p, W2_ref[...], preferred_element_type=jnp.float32)
    h = h + b2_ref[...] + emb_ref[...].astype(jnp.float32)

    j = jnp.full((_D, _D), 1.0 / _D, jnp.float32)
    mean = jnp.dot(h, j, preferred_element_type=jnp.float32)
    c = h - mean
    var = jnp.dot(c * c, j, preferred_element_type=jnp.float32)
    out_ref[...] = c * jax.lax.rsqrt(var + 1e-5) * g_ref[...] + bt_ref[...]


def _tc_fused(emb, x, W1, b1, W2, b2, gamma, beta):
    grid = (_TOK // _T,)
    return pl.pallas_call(
        _tc_body,
        grid=grid,
        in_specs=[
            pl.BlockSpec((_T, _D), lambda i: (i, 0)),
            pl.BlockSpec((_T, 3), lambda i: (i, 0)),
            pl.BlockSpec((3, _D), lambda i: (0, 0)),
            pl.BlockSpec((1, _D), lambda i: (0, 0)),
            pl.BlockSpec((_D, _D), lambda i: (0, 0)),
            pl.BlockSpec((1, _D), lambda i: (0, 0)),
            pl.BlockSpec((1, _D), lambda i: (0, 0)),
            pl.BlockSpec((1, _D), lambda i: (0, 0)),
        ],
        out_specs=pl.BlockSpec((_T, _D), lambda i: (i, 0)),
        out_shape=jax.ShapeDtypeStruct((_TOK, _D), jnp.float32),
        compiler_params=pltpu.CompilerParams(
            dimension_semantics=("arbitrary",),
        ),
    )(emb, x, W1, b1, W2, b2, gamma, beta)


@jax.jit
def _run(z, x, table, W1, b1, W2, b2, gamma, beta):
    table_w = jax.lax.bitcast_convert_type(
        table.astype(jnp.bfloat16).reshape(_NTYPES, _D // 2, 2),
        jnp.int32)
    emb_w = _sc_gather(z, table_w)
    emb = jax.lax.bitcast_convert_type(emb_w, jnp.bfloat16)
    emb = emb.reshape(_TOK, _D)
    return _tc_fused(emb, x, W1, b1, W2, b2, gamma, beta)


def kernel(z, x, cu_seqlens, table, W1, b1, W2, b2, gamma, beta):
    del cu_seqlens
    return _run(z.astype(jnp.int32), x, table, W1,
                b1.reshape(1, _D), W2, b2.reshape(1, _D),
                gamma.reshape(1, _D), beta.reshape(1, _D))


# speedup vs baseline: 2.4910x; 2.4910x over previous
import functools

import jax
import jax.numpy as jnp
from jax import lax
from jax.experimental import pallas as pl
from jax.experimental.pallas import tpu as pltpu
from jax.experimental.pallas import tpu_sc as plsc

_TOK = 32768
_D = 128
_T = 8192

_NC = 2
_NS = 16
_NW = _NC * _NS
_BPW = _TOK // _NW
_CH = 128
_NCH = _BPW // _CH
_NTYPES = 100


def _sc_body(z2_hbm, table_hbm, out_hbm, idx_v, rows_a, rows_b, tab_sh,
             bounce, gsem, ssem_a, ssem_b):
    cid = lax.axis_index("c")
    sid = lax.axis_index("s")
    wid = sid * _NC + cid
    base = wid * _BPW
    brow = wid * _NCH

    @pl.when(sid == 0)
    def _():
        pltpu.sync_copy(table_hbm, bounce)
        pltpu.sync_copy(bounce, tab_sh)
    plsc.subcore_barrier()

    pltpu.sync_copy(z2_hbm.at[pl.ds(brow, _NCH)], idx_v)

    bufs = (rows_a, rows_b)
    ssems = (ssem_a, ssem_b)
    stores = [None, None]
    gath = pltpu.async_copy(tab_sh.at[idx_v.at[0]], bufs[0], gsem)
    for i in range(_NCH):
        b = i % 2
        gath.wait()
        st = pltpu.async_copy(
            bufs[b], out_hbm.at[pl.ds(base + i * _CH, _CH)], ssems[b])
        stores[b] = st
        if i + 1 < _NCH:
            nb = (i + 1) % 2
            if stores[nb] is not None:
                stores[nb].wait()
            gath = pltpu.async_copy(
                tab_sh.at[idx_v.at[i + 1]], bufs[nb], gsem)
    stores[(_NCH - 2) % 2].wait()
    stores[(_NCH - 1) % 2].wait()


def _sc_gather(z, table):
    mesh = plsc.VectorSubcoreMesh(core_axis_name="c", subcore_axis_name="s")
    return pl.kernel(
        _sc_body,
        mesh=mesh,
        out_type=jax.ShapeDtypeStruct((_TOK, _D), jnp.float32),
        scratch_types=[
            pltpu.VMEM((_NCH, _CH), jnp.int32),
            pltpu.VMEM((_CH, _D), jnp.float32),
            pltpu.VMEM((_CH, _D), jnp.float32),
            pltpu.VMEM_SHARED((_NTYPES, _D), jnp.float32),
            pltpu.VMEM((_NTYPES, _D), jnp.float32),
            pltpu.SemaphoreType.DMA,
            pltpu.SemaphoreType.DMA,
            pltpu.SemaphoreType.DMA,
        ],
    )(z.reshape(_TOK // _CH, _CH), table)


def _tc_body(emb_ref, x_ref, W1_ref, b1_ref, W2_ref, b2_ref,
             g_ref, bt_ref, out_ref):
    x = x_ref[...]
    p = jnp.dot(x, W1_ref[...], preferred_element_type=jnp.float32)
    p = p + b1_ref[...]
    p = p * jax.nn.sigmoid(p)
    h = jnp.dot(p, W2_ref[...], preferred_element_type=jnp.float32)
    h = h + b2_ref[...] + emb_ref[...]

    j = jnp.full((_D, _D), 1.0 / _D, jnp.float32)
    mean = jnp.dot(h, j, preferred_element_type=jnp.float32)
    c = h - mean
    var = jnp.dot(c * c, j, preferred_element_type=jnp.float32)
    out_ref[...] = c * jax.lax.rsqrt(var + 1e-5) * g_ref[...] + bt_ref[...]


def _tc_fused(emb, x, W1, b1, W2, b2, gamma, beta):
    grid = (_TOK // _T,)
    return pl.pallas_call(
        _tc_body,
        grid=grid,
        in_specs=[
            pl.BlockSpec((_T, _D), lambda i: (i, 0)),
            pl.BlockSpec((_T, 3), lambda i: (i, 0)),
            pl.BlockSpec((3, _D), lambda i: (0, 0)),
            pl.BlockSpec((1, _D), lambda i: (0, 0)),
            pl.BlockSpec((_D, _D), lambda i: (0, 0)),
            pl.BlockSpec((1, _D), lambda i: (0, 0)),
            pl.BlockSpec((1, _D), lambda i: (0, 0)),
            pl.BlockSpec((1, _D), lambda i: (0, 0)),
        ],
        out_specs=pl.BlockSpec((_T, _D), lambda i: (i, 0)),
        out_shape=jax.ShapeDtypeStruct((_TOK, _D), jnp.float32),
        compiler_params=pltpu.CompilerParams(
            dimension_semantics=("arbitrary",),
        ),
    )(emb, x, W1, b1, W2, b2, gamma, beta)


@jax.jit
def _run(z, x, table, W1, b1, W2, b2, gamma, beta):
    emb = _sc_gather(z, table)
    return _tc_fused(emb, x, W1, b1, W2, b2, gamma, beta)


def kernel(z, x, cu_seqlens, table, W1, b1, W2, b2, gamma, beta):
    del cu_seqlens
    return _run(z.astype(jnp.int32), x, table, W1,
                b1.reshape(1, _D), W2, b2.reshape(1, _D),
                gamma.reshape(1, _D), beta.reshape(1, _D))


# speedup vs baseline: 2.4936x; 1.0010x over previous
import jax
import jax.numpy as jnp
from jax import lax
from jax.experimental import pallas as pl
from jax.experimental.pallas import tpu as pltpu
from jax.experimental.pallas import tpu_sc as plsc

_TOK = 32768
_D = 128
_T = 8192

_NC = 2
_NS = 16
_NW = _NC * _NS
_BPW = _TOK // _NW
_CH = 128
_NCH = _BPW // _CH
_NTYPES = 100


def _sc_body(z2_hbm, table_hbm, out_hbm, idx_v, rows_a, rows_b, tab_sh,
             bounce, gsem, ssem_a, ssem_b):
    cid = lax.axis_index("c")
    sid = lax.axis_index("s")
    wid = sid * _NC + cid
    base = wid * _BPW
    brow = wid * _NCH

    @pl.when(sid == 0)
    def _():
        pltpu.sync_copy(table_hbm, bounce)
        pltpu.sync_copy(bounce, tab_sh)
    plsc.subcore_barrier()

    pltpu.sync_copy(z2_hbm.at[pl.ds(brow, _NCH)], idx_v)

    bufs = (rows_a, rows_b)
    ssems = (ssem_a, ssem_b)
    stores = [None, None]
    gath = pltpu.async_copy(tab_sh.at[idx_v.at[0]], bufs[0], gsem)
    for i in range(_NCH):
        b = i % 2
        gath.wait()
        st = pltpu.async_copy(
            bufs[b], out_hbm.at[pl.ds(base + i * _CH, _CH)], ssems[b])
        stores[b] = st
        if i + 1 < _NCH:
            nb = (i + 1) % 2
            if stores[nb] is not None:
                stores[nb].wait()
            gath = pltpu.async_copy(
                tab_sh.at[idx_v.at[i + 1]], bufs[nb], gsem)
    stores[(_NCH - 2) % 2].wait()
    stores[(_NCH - 1) % 2].wait()


def _sc_gather(z, table):
    mesh = plsc.VectorSubcoreMesh(core_axis_name="c", subcore_axis_name="s")
    return pl.kernel(
        _sc_body,
        mesh=mesh,
        out_type=jax.ShapeDtypeStruct((_TOK, _D), jnp.float32),
        scratch_types=[
            pltpu.VMEM((_NCH, _CH), jnp.int32),
            pltpu.VMEM((_CH, _D), jnp.float32),
            pltpu.VMEM((_CH, _D), jnp.float32),
            pltpu.VMEM_SHARED((_NTYPES, _D), jnp.float32),
            pltpu.VMEM((_NTYPES, _D), jnp.float32),
            pltpu.SemaphoreType.DMA,
            pltpu.SemaphoreType.DMA,
            pltpu.SemaphoreType.DMA,
        ],
    )(z.reshape(_TOK // _CH, _CH), table)


def _tc_body(emb_ref, x_ref, W1_ref, b1_ref, W2_ref, b2_ref,
             g_ref, bt_ref, out_ref):
    x = x_ref[...]
    p = jnp.dot(x, W1_ref[...], preferred_element_type=jnp.float32)
    p = p + b1_ref[...]
    p = p * jax.nn.sigmoid(p)
    h = jnp.dot(p, W2_ref[...], preferred_element_type=jnp.float32)
    h = h + b2_ref[...] + emb_ref[...]

    j = jnp.full((_D, _D), 1.0 / _D, jnp.float32)
    mean = jnp.dot(h, j, preferred_element_type=jnp.float32)
    c = h - mean
    var = jnp.dot(c * c, j, preferred_element_type=jnp.float32)
    out_ref[...] = c * jax.lax.rsqrt(var + 1e-5) * g_ref[...] + bt_ref[...]


def _tc_fused(emb, x, W1, b1, W2, b2, gamma, beta):
    grid = (_TOK // _T,)
    return pl.pallas_call(
        _tc_body,
        grid=grid,
        in_specs=[
            pl.BlockSpec((_T, _D), lambda i: (i, 0)),
            pl.BlockSpec((_T, 3), lambda i: (i, 0)),
            pl.BlockSpec((3, _D), lambda i: (0, 0)),
            pl.BlockSpec((1, _D), lambda i: (0, 0)),
            pl.BlockSpec((_D, _D), lambda i: (0, 0)),
            pl.BlockSpec((1, _D), lambda i: (0, 0)),
            pl.BlockSpec((1, _D), lambda i: (0, 0)),
            pl.BlockSpec((1, _D), lambda i: (0, 0)),
        ],
        out_specs=pl.BlockSpec((_T, _D), lambda i: (i, 0)),
        out_shape=jax.ShapeDtypeStruct((_TOK, _D), jnp.float32),
        compiler_params=pltpu.CompilerParams(
            dimension_semantics=("arbitrary",),
        ),
    )(emb, x, W1, b1, W2, b2, gamma, beta)


@jax.jit
def _run(z, x, table, W1, b1, W2, b2, gamma, beta):
    emb = _sc_gather(z, table)
    return _tc_fused(emb, x, W1, b1, W2, b2, gamma, beta)


def kernel(z, x, cu_seqlens, table, W1, b1, W2, b2, gamma, beta):
    del cu_seqlens
    return _run(z.astype(jnp.int32), x, table, W1,
                b1.reshape(1, _D), W2, b2.reshape(1, _D),
                gamma.reshape(1, _D), beta.reshape(1, _D))
